# baseline (speedup 1.0000x reference)
import functools

import jax
import jax.numpy as jnp
from jax import lax
from jax.experimental import pallas as pl
from jax.experimental.pallas import tpu as pltpu
from jax.experimental.pallas import tpu_sc as plsc

N = 10000
NP = 10240
E = 320000
D = 128
EPS = 1e-5

RPT = 320
NTR = 336
CAP = 11520
STG = 16000
NSTG = E // STG
CH = 128
NCHA = CAP // CH
TRASH = 328

_MESH = plsc.VectorSubcoreMesh(core_axis_name="c", subcore_axis_name="s")
_SC_PARAMS = pltpu.CompilerParams(needs_layout_passes=False)


@functools.partial(
    pl.kernel,
    out_type=(jax.ShapeDtypeStruct((32, CAP), jnp.int32),
              jax.ShapeDtypeStruct((32, CAP), jnp.int32),
              jax.ShapeDtypeStruct((32, NTR), jnp.float32)),
    mesh=_MESH,
    compiler_params=_SC_PARAMS,
    scratch_types=[pltpu.VMEM((STG,), jnp.int32),
                   pltpu.VMEM((STG,), jnp.int32),
                   pltpu.VMEM((CAP,), jnp.int32),
                   pltpu.VMEM((CAP,), jnp.int32),
                   pltpu.VMEM((NTR,), jnp.float32)])
def _sc_route(src_hbm, dst_hbm, srcl_hbm, offl_hbm, deg_hbm,
              stage_src, stage_dst, slist, olist, deg_v):
    c = lax.axis_index("c")
    s = lax.axis_index("s")
    w = c * 16 + s
    base = w * RPT

    def prefill(i, _):
        slist[pl.ds(i * 16, 16)] = jnp.full((16,), N, jnp.int32)
        olist[pl.ds(i * 16, 16)] = jnp.full((16,), TRASH, jnp.int32)
        return 0
    lax.fori_loop(0, CAP // 16, prefill, 0, unroll=False)

    def zdeg(i, _):
        deg_v[pl.ds(i * 16, 16)] = jnp.zeros((16,), jnp.float32)
        return 0
    lax.fori_loop(0, NTR // 16, zdeg, 0, unroll=False)

    def outer(t, cnt):
        pltpu.sync_copy(src_hbm.at[pl.ds(t * STG, STG)], stage_src)
        pltpu.sync_copy(dst_hbm.at[pl.ds(t * STG, STG)], stage_dst)

        def inner(i, cnt):
            sv = stage_src[pl.ds(i * 16, 16)]
            dv = stage_dst[pl.ds(i * 16, 16)]
            off = dv - base
            msk = (off >= 0) & (off < RPT)
            inc = msk.astype(jnp.int32)
            offc = jnp.where(msk, off, TRASH)
            packed = jnp.where(msk, sv * 512 + off, N * 512 + TRASH)
            _, pk = plsc.sort_key_val(1 - inc, packed)
            slist[pl.ds(cnt, 16)] = pk >> 9
            olist[pl.ds(cnt, 16)] = pk & 511
            plsc.addupdate_scatter(deg_v, [offc], jnp.where(msk, 1.0, 0.0))
            n = (inc[0] + inc[1] + inc[2] + inc[3] + inc[4] + inc[5] + inc[6]
                 + inc[7] + inc[8] + inc[9] + inc[10] + inc[11] + inc[12]
                 + inc[13] + inc[14] + inc[15])
            return jnp.minimum(cnt + n, CAP - 16)

        return lax.fori_loop(0, STG // 16, inner, cnt, unroll=False)

    lax.fori_loop(0, NSTG, outer, 0, unroll=False)
    pltpu.sync_copy(slist, srcl_hbm.at[w])
    pltpu.sync_copy(olist, offl_hbm.at[w])
    pltpu.sync_copy(deg_v, deg_hbm.at[w])


@functools.partial(
    pl.kernel,
    out_type=jax.ShapeDtypeStruct((32, RPT * D), jnp.float32),
    mesh=_MESH,
    compiler_params=_SC_PARAMS,
    scratch_types=[pltpu.VMEM((NCHA, CH), jnp.int32),
                   pltpu.VMEM((CAP,), jnp.int32),
                   pltpu.VMEM((CH, D), jnp.float32),
                   pltpu.VMEM((NTR * D,), jnp.float32),
                   pltpu.SemaphoreType.DMA])
def _sc_accum(srcl_hbm, offl_hbm, g_hbm, z_hbm, out_hbm,
              src_v, off_v, rows_v, acc_f, sem):
    c = lax.axis_index("c")
    s = lax.axis_index("s")
    w = c * 16 + s
    pltpu.sync_copy(srcl_hbm.at[w], src_v)
    pltpu.sync_copy(offl_hbm.at[w], off_v)
    pltpu.sync_copy(z_hbm, acc_f.at[pl.ds(0, 16384)])
    pltpu.sync_copy(z_hbm, acc_f.at[pl.ds(16384, 16384)])
    pltpu.sync_copy(z_hbm.at[pl.ds(0, 10240)], acc_f.at[pl.ds(32768, 10240)])

    def chunk(j, _):
        pltpu.async_copy(g_hbm.at[src_v.at[j]], rows_v, sem).wait()

        def group(q, _):
            offs = off_v[pl.ds(j * CH + q * 16, 16)]
            for e in range(16):
                rb = offs[e] * D
                sb = (q * 16 + e) * D
                for k in range(8):
                    plsc.addupdate(acc_f.at[pl.ds(rb + k * 16, 16)],
                                   rows_v[q * 16 + e, pl.ds(k * 16, 16)])
            return 0

        lax.fori_loop(0, CH // 16, group, 0, unroll=False)
        return 0

    lax.fori_loop(0, NCHA, chunk, 0, unroll=False)
    pltpu.sync_copy(acc_f.at[pl.ds(0, RPT * D)], out_hbm.at[w])


def _tc1_body(deg_ref, x_ref, w1_ref, g_ref, dinv_ref):
    deg = deg_ref[...] + 1.0
    dinv = lax.rsqrt(deg)
    rows = lax.broadcasted_iota(jnp.int32, (NP, 1), 0)
    dinv = jnp.where(rows < N, dinv, 0.0)
    dinv_ref[...] = dinv
    h = jnp.dot(x_ref[...], w1_ref[...], preferred_element_type=jnp.float32)
    g_ref[...] = h * dinv


_tc1 = pl.pallas_call(
    _tc1_body,
    out_shape=(
        jax.ShapeDtypeStruct((NP, D), jnp.float32),
        jax.ShapeDtypeStruct((NP, 1), jnp.float32),
    ),
)


def _tc2_body(s_ref, g_ref, dinv_ref, b_ref, gam_ref, bet_ref, w2_ref, out_ref):
    dinv = dinv_ref[...]
    h = dinv * (s_ref[...] + g_ref[...]) + b_ref[...]
    rows = lax.broadcasted_iota(jnp.int32, (NP, 1), 0)
    m = rows < N
    hm = jnp.where(m, h, 0.0)
    mean = jnp.sum(hm, axis=0, keepdims=True) * (1.0 / N)
    cen = h - mean
    cenm = jnp.where(m, cen, 0.0)
    var = jnp.sum(cenm * cenm, axis=0, keepdims=True) * (1.0 / N)
    hbn = cen * lax.rsqrt(var + EPS) * gam_ref[...] + bet_ref[...]
    hr = jnp.maximum(hbn, 0.0)
    h2 = jnp.dot(hr, w2_ref[...], preferred_element_type=jnp.float32)
    out_ref[...] = jnp.where(m, h2 * dinv, 0.0)


_tc2 = pl.pallas_call(
    _tc2_body,
    out_shape=jax.ShapeDtypeStruct((NP, D), jnp.float32),
)


def _tc3_body(s_ref, g_ref, dinv_ref, b_ref, gam_ref, bet_ref,
              wm1_ref, bm1_ref, wm2_ref, bm2_ref, out_ref):
    dinv = dinv_ref[...]
    h = dinv * (s_ref[...] + g_ref[...]) + b_ref[...]
    rows = lax.broadcasted_iota(jnp.int32, (NP, 1), 0)
    m = rows < N
    hm = jnp.where(m, h, 0.0)
    mean = jnp.sum(hm, axis=0, keepdims=True) * (1.0 / N)
    cen = h - mean
    cenm = jnp.where(m, cen, 0.0)
    var = jnp.sum(cenm * cenm, axis=0, keepdims=True) * (1.0 / N)
    hbn = cen * lax.rsqrt(var + EPS) * gam_ref[...] + bet_ref[...]
    hr = jnp.maximum(hbn, 0.0)
    z = jnp.dot(hr, wm1_ref[...], preferred_element_type=jnp.float32)
    z = jnp.maximum(z + bm1_ref[...], 0.0)
    y = jnp.dot(z, wm2_ref[...], preferred_element_type=jnp.float32)
    out_ref[...] = y + bm2_ref[...]


_tc3 = pl.pallas_call(
    _tc3_body,
    out_shape=jax.ShapeDtypeStruct((NP, 1), jnp.float32),
)


def kernel(x, edge_index, W1, b1, gamma1, beta1, W2, b2, gamma2, beta2,
           Wm1, bm1, Wm2, bm2):
    ei = edge_index.astype(jnp.int32)
    src, dst = ei[0], ei[1]
    x_p = jnp.pad(x, ((0, NP - N), (0, 0)))
    zflat = jnp.zeros((16384,), jnp.float32)

    srcl, offl, degt = _sc_route(src, dst)
    deg = degt[:, :RPT].reshape(NP, 1)
    srcl3 = srcl.reshape(32, NCHA, CH)

    g1, dinv = _tc1(deg, x_p, W1)
    s1 = _sc_accum(srcl3, offl, g1, zflat).reshape(NP, D)
    g2 = _tc2(s1, g1, dinv, b1.reshape(1, D), gamma1.reshape(1, D),
              beta1.reshape(1, D), W2)
    s2 = _sc_accum(srcl3, offl, g2, zflat).reshape(NP, D)
    y = _tc3(s2, g2, dinv, b2.reshape(1, D), gamma2.reshape(1, D),
             beta2.reshape(1, D), Wm1, bm1.reshape(1, 64), Wm2,
             bm2.reshape(1, 1))
    return y[:N, 0]


# speedup vs baseline: 1.0843x; 1.0843x over previous
import functools

import jax
import jax.numpy as jnp
from jax import lax
from jax.experimental import pallas as pl
from jax.experimental.pallas import tpu as pltpu
from jax.experimental.pallas import tpu_sc as plsc

N = 10000
NP = 10240
E = 320000
D = 128
EPS = 1e-5

RPT = 320
NTR = 336
CAP = 11520
STG = 16000
NSTG = E // STG
CH = 128
NCHA = CAP // CH
TRASH = 328

_MESH = plsc.VectorSubcoreMesh(core_axis_name="c", subcore_axis_name="s")
_SC_PARAMS = pltpu.CompilerParams(needs_layout_passes=False)


@functools.partial(
    pl.kernel,
    out_type=(jax.ShapeDtypeStruct((32, CAP), jnp.int32),
              jax.ShapeDtypeStruct((32, CAP), jnp.int32),
              jax.ShapeDtypeStruct((32, NTR), jnp.float32)),
    mesh=_MESH,
    compiler_params=_SC_PARAMS,
    scratch_types=[pltpu.VMEM((STG,), jnp.int32),
                   pltpu.VMEM((STG,), jnp.int32),
                   pltpu.VMEM((CAP,), jnp.int32),
                   pltpu.VMEM((CAP,), jnp.int32),
                   pltpu.VMEM((NTR,), jnp.float32)])
def _sc_route(src_hbm, dst_hbm, srcl_hbm, offl_hbm, deg_hbm,
              stage_src, stage_dst, slist, olist, deg_v):
    c = lax.axis_index("c")
    s = lax.axis_index("s")
    w = c * 16 + s
    base = w * RPT

    def prefill(i, _):
        slist[pl.ds(i * 16, 16)] = jnp.full((16,), N, jnp.int32)
        olist[pl.ds(i * 16, 16)] = jnp.full((16,), TRASH, jnp.int32)
        return 0
    lax.fori_loop(0, CAP // 16, prefill, 0, unroll=False)

    def zdeg(i, _):
        deg_v[pl.ds(i * 16, 16)] = jnp.zeros((16,), jnp.float32)
        return 0
    lax.fori_loop(0, NTR // 16, zdeg, 0, unroll=False)

    def outer(t, cnt):
        pltpu.sync_copy(src_hbm.at[pl.ds(t * STG, STG)], stage_src)
        pltpu.sync_copy(dst_hbm.at[pl.ds(t * STG, STG)], stage_dst)

        def inner(i, cnt):
            sv = stage_src[pl.ds(i * 16, 16)]
            dv = stage_dst[pl.ds(i * 16, 16)]
            off = dv - base
            msk = (off >= 0) & (off < RPT)
            inc = msk.astype(jnp.int32)
            offc = jnp.where(msk, off, TRASH)
            packed = jnp.where(msk, sv * 512 + off, N * 512 + TRASH)
            _, pk = plsc.sort_key_val(1 - inc, packed)
            slist[pl.ds(cnt, 16)] = pk >> 9
            olist[pl.ds(cnt, 16)] = pk & 511
            plsc.addupdate_scatter(deg_v, [offc], jnp.where(msk, 1.0, 0.0))
            n = (inc[0] + inc[1] + inc[2] + inc[3] + inc[4] + inc[5] + inc[6]
                 + inc[7] + inc[8] + inc[9] + inc[10] + inc[11] + inc[12]
                 + inc[13] + inc[14] + inc[15])
            return jnp.minimum(cnt + n, CAP - 16)

        return lax.fori_loop(0, STG // 16, inner, cnt, unroll=False)

    lax.fori_loop(0, NSTG, outer, 0, unroll=False)
    pltpu.sync_copy(slist, srcl_hbm.at[w])
    pltpu.sync_copy(olist, offl_hbm.at[w])
    pltpu.sync_copy(deg_v, deg_hbm.at[w])


@functools.partial(
    pl.kernel,
    out_type=jax.ShapeDtypeStruct((32, RPT * D), jnp.float32),
    mesh=_MESH,
    compiler_params=_SC_PARAMS,
    scratch_types=[pltpu.VMEM((NCHA, CH), jnp.int32),
                   pltpu.VMEM((CAP,), jnp.int32),
                   pltpu.VMEM((CH, D), jnp.float32),
                   pltpu.VMEM((CH, D), jnp.float32),
                   pltpu.VMEM((NTR * D,), jnp.float32),
                   pltpu.SemaphoreType.DMA,
                   pltpu.SemaphoreType.DMA])
def _sc_accum(srcl_hbm, offl_hbm, g_hbm, z_hbm, out_hbm,
              src_v, off_v, rows_a, rows_b, acc_f, sem_a, sem_b):
    c = lax.axis_index("c")
    s = lax.axis_index("s")
    w = c * 16 + s
    pltpu.sync_copy(srcl_hbm.at[w], src_v)
    pltpu.sync_copy(offl_hbm.at[w], off_v)
    pltpu.sync_copy(z_hbm, acc_f.at[pl.ds(0, 16384)])
    pltpu.sync_copy(z_hbm, acc_f.at[pl.ds(16384, 16384)])
    pltpu.sync_copy(z_hbm.at[pl.ds(0, 10240)], acc_f.at[pl.ds(32768, 10240)])

    def process(j, rows_v):
        def group(q, _):
            offs = off_v[pl.ds(j * CH + q * 16, 16)]
            for e in range(16):
                rb = offs[e] * D
                for k in range(8):
                    plsc.addupdate(acc_f.at[pl.ds(rb + k * 16, 16)],
                                   rows_v[q * 16 + e, pl.ds(k * 16, 16)])
            return 0

        lax.fori_loop(0, CH // 16, group, 0, unroll=False)

    pltpu.async_copy(g_hbm.at[src_v.at[0]], rows_a, sem_a)

    def pair(p, _):
        ja = 2 * p
        jb = 2 * p + 1
        pltpu.make_async_copy(g_hbm.at[src_v.at[ja]], rows_a, sem_a).wait()
        pltpu.async_copy(g_hbm.at[src_v.at[jb]], rows_b, sem_b)
        process(ja, rows_a)
        pltpu.make_async_copy(g_hbm.at[src_v.at[jb]], rows_b, sem_b).wait()

        @pl.when(jb + 1 < NCHA)
        def _():
            pltpu.async_copy(g_hbm.at[src_v.at[jb + 1]], rows_a, sem_a)

        process(jb, rows_b)
        return 0

    lax.fori_loop(0, NCHA // 2, pair, 0, unroll=False)
    pltpu.sync_copy(acc_f.at[pl.ds(0, RPT * D)], out_hbm.at[w])


def _tc1_body(deg_ref, x_ref, w1_ref, g_ref, dinv_ref):
    deg = deg_ref[...] + 1.0
    dinv = lax.rsqrt(deg)
    rows = lax.broadcasted_iota(jnp.int32, (NP, 1), 0)
    dinv = jnp.where(rows < N, dinv, 0.0)
    dinv_ref[...] = dinv
    h = jnp.dot(x_ref[...], w1_ref[...], preferred_element_type=jnp.float32)
    g_ref[...] = h * dinv


_tc1 = pl.pallas_call(
    _tc1_body,
    out_shape=(
        jax.ShapeDtypeStruct((NP, D), jnp.float32),
        jax.ShapeDtypeStruct((NP, 1), jnp.float32),
    ),
)


def _tc2_body(s_ref, g_ref, dinv_ref, b_ref, gam_ref, bet_ref, w2_ref, out_ref):
    dinv = dinv_ref[...]
    h = dinv * (s_ref[...] + g_ref[...]) + b_ref[...]
    rows = lax.broadcasted_iota(jnp.int32, (NP, 1), 0)
    m = rows < N
    hm = jnp.where(m, h, 0.0)
    mean = jnp.sum(hm, axis=0, keepdims=True) * (1.0 / N)
    cen = h - mean
    cenm = jnp.where(m, cen, 0.0)
    var = jnp.sum(cenm * cenm, axis=0, keepdims=True) * (1.0 / N)
    hbn = cen * lax.rsqrt(var + EPS) * gam_ref[...] + bet_ref[...]
    hr = jnp.maximum(hbn, 0.0)
    h2 = jnp.dot(hr, w2_ref[...], preferred_element_type=jnp.float32)
    out_ref[...] = jnp.where(m, h2 * dinv, 0.0)


_tc2 = pl.pallas_call(
    _tc2_body,
    out_shape=jax.ShapeDtypeStruct((NP, D), jnp.float32),
)


def _tc3_body(s_ref, g_ref, dinv_ref, b_ref, gam_ref, bet_ref,
              wm1_ref, bm1_ref, wm2_ref, bm2_ref, out_ref):
    dinv = dinv_ref[...]
    h = dinv * (s_ref[...] + g_ref[...]) + b_ref[...]
    rows = lax.broadcasted_iota(jnp.int32, (NP, 1), 0)
    m = rows < N
    hm = jnp.where(m, h, 0.0)
    mean = jnp.sum(hm, axis=0, keepdims=True) * (1.0 / N)
    cen = h - mean
    cenm = jnp.where(m, cen, 0.0)
    var = jnp.sum(cenm * cenm, axis=0, keepdims=True) * (1.0 / N)
    hbn = cen * lax.rsqrt(var + EPS) * gam_ref[...] + bet_ref[...]
    hr = jnp.maximum(hbn, 0.0)
    z = jnp.dot(hr, wm1_ref[...], preferred_element_type=jnp.float32)
    z = jnp.maximum(z + bm1_ref[...], 0.0)
    y = jnp.dot(z, wm2_ref[...], preferred_element_type=jnp.float32)
    out_ref[...] = y + bm2_ref[...]


_tc3 = pl.pallas_call(
    _tc3_body,
    out_shape=jax.ShapeDtypeStruct((NP, 1), jnp.float32),
)


def kernel(x, edge_index, W1, b1, gamma1, beta1, W2, b2, gamma2, beta2,
           Wm1, bm1, Wm2, bm2):
    ei = edge_index.astype(jnp.int32)
    src, dst = ei[0], ei[1]
    x_p = jnp.pad(x, ((0, NP - N), (0, 0)))
    zflat = jnp.zeros((16384,), jnp.float32)

    srcl, offl, degt = _sc_route(src, dst)
    deg = degt[:, :RPT].reshape(NP, 1)
    srcl3 = srcl.reshape(32, NCHA, CH)

    g1, dinv = _tc1(deg, x_p, W1)
    s1 = _sc_accum(srcl3, offl, g1, zflat).reshape(NP, D)
    g2 = _tc2(s1, g1, dinv, b1.reshape(1, D), gamma1.reshape(1, D),
              beta1.reshape(1, D), W2)
    s2 = _sc_accum(srcl3, offl, g2, zflat).reshape(NP, D)
    y = _tc3(s2, g2, dinv, b2.reshape(1, D), gamma2.reshape(1, D),
             beta2.reshape(1, D), Wm1, bm1.reshape(1, 64), Wm2,
             bm2.reshape(1, 1))
    return y[:N, 0]


# speedup vs baseline: 1.4463x; 1.3339x over previous
import functools

import jax
import jax.numpy as jnp
from jax import lax
from jax.experimental import pallas as pl
from jax.experimental.pallas import tpu as pltpu
from jax.experimental.pallas import tpu_sc as plsc

N = 10000
NP = 10240
E = 320000
D = 128
EPS = 1e-5

RPT = 320
NTR = 336
CAP = 11008
STG = 16000
NSTG = E // STG
CH = 128
NCHA = CAP // CH
TRASH = 328

_MESH = plsc.VectorSubcoreMesh(core_axis_name="c", subcore_axis_name="s")
_SC_PARAMS = pltpu.CompilerParams(needs_layout_passes=False)


@functools.partial(
    pl.kernel,
    out_type=(jax.ShapeDtypeStruct((32, CAP), jnp.int32),
              jax.ShapeDtypeStruct((32, CAP), jnp.int32),
              jax.ShapeDtypeStruct((32, NTR), jnp.float32)),
    mesh=_MESH,
    compiler_params=_SC_PARAMS,
    scratch_types=[pltpu.VMEM((STG,), jnp.int32),
                   pltpu.VMEM((STG,), jnp.int32),
                   pltpu.VMEM((CAP,), jnp.int32),
                   pltpu.VMEM((CAP,), jnp.int32),
                   pltpu.VMEM((NTR,), jnp.float32)])
def _sc_route(src_hbm, dst_hbm, srcl_hbm, offl_hbm, deg_hbm,
              stage_src, stage_dst, slist, olist, deg_v):
    c = lax.axis_index("c")
    s = lax.axis_index("s")
    w = c * 16 + s
    base = w * RPT

    def prefill(i, _):
        slist[pl.ds(i * 16, 16)] = jnp.full((16,), N, jnp.int32)
        olist[pl.ds(i * 16, 16)] = jnp.full((16,), TRASH, jnp.int32)
        return 0
    lax.fori_loop(0, CAP // 16, prefill, 0, unroll=False)

    def zdeg(i, _):
        deg_v[pl.ds(i * 16, 16)] = jnp.zeros((16,), jnp.float32)
        return 0
    lax.fori_loop(0, NTR // 16, zdeg, 0, unroll=False)

    def outer(t, cnt):
        pltpu.sync_copy(src_hbm.at[pl.ds(t * STG, STG)], stage_src)
        pltpu.sync_copy(dst_hbm.at[pl.ds(t * STG, STG)], stage_dst)

        def inner(i, cnt):
            sv = stage_src[pl.ds(i * 16, 16)]
            dv = stage_dst[pl.ds(i * 16, 16)]
            off = dv - base
            msk = (off >= 0) & (off < RPT)
            inc = msk.astype(jnp.int32)
            offc = jnp.where(msk, off, TRASH)
            packed = jnp.where(msk, sv * 512 + off, N * 512 + TRASH)
            _, pk = plsc.sort_key_val(1 - inc, packed)
            slist[pl.ds(cnt, 16)] = pk >> 9
            olist[pl.ds(cnt, 16)] = pk & 511
            plsc.addupdate_scatter(deg_v, [offc], jnp.where(msk, 1.0, 0.0))
            n = plsc.all_reduce_population_count(msk)[0]
            return jnp.minimum(cnt + n, CAP - 16)

        return lax.fori_loop(0, STG // 16, inner, cnt, unroll=False)

    lax.fori_loop(0, NSTG, outer, 0, unroll=False)
    pltpu.sync_copy(slist, srcl_hbm.at[w])
    pltpu.sync_copy(olist, offl_hbm.at[w])
    pltpu.sync_copy(deg_v, deg_hbm.at[w])


@functools.partial(
    pl.kernel,
    out_type=jax.ShapeDtypeStruct((32, RPT * D), jnp.float32),
    mesh=_MESH,
    compiler_params=_SC_PARAMS,
    scratch_types=[pltpu.VMEM((NCHA, CH), jnp.int32),
                   pltpu.VMEM((CAP,), jnp.int32),
                   pltpu.VMEM((CH, D), jnp.float32),
                   pltpu.VMEM((CH, D), jnp.float32),
                   pltpu.VMEM((NTR * D,), jnp.float32),
                   pltpu.SemaphoreType.DMA,
                   pltpu.SemaphoreType.DMA])
def _sc_accum(srcl_hbm, offl_hbm, g_hbm, z_hbm, out_hbm,
              src_v, off_v, rows_a, rows_b, acc_f, sem_a, sem_b):
    c = lax.axis_index("c")
    s = lax.axis_index("s")
    w = c * 16 + s
    pltpu.sync_copy(srcl_hbm.at[w], src_v)
    pltpu.sync_copy(offl_hbm.at[w], off_v)
    pltpu.sync_copy(z_hbm, acc_f.at[pl.ds(0, 16384)])
    pltpu.sync_copy(z_hbm, acc_f.at[pl.ds(16384, 16384)])
    pltpu.sync_copy(z_hbm.at[pl.ds(0, 10240)], acc_f.at[pl.ds(32768, 10240)])

    def process(j, rows_v):
        def group(q, _):
            offs = off_v[pl.ds(j * CH + q * 16, 16)]
            for e in range(16):
                rb = offs[e] * D
                for k in range(8):
                    plsc.addupdate(acc_f.at[pl.ds(rb + k * 16, 16)],
                                   rows_v[q * 16 + e, pl.ds(k * 16, 16)])
            return 0

        lax.fori_loop(0, CH // 16, group, 0, unroll=False)

    pltpu.async_copy(g_hbm.at[src_v.at[0]], rows_a, sem_a)

    def pair(p, _):
        ja = 2 * p
        jb = 2 * p + 1
        pltpu.make_async_copy(g_hbm.at[src_v.at[ja]], rows_a, sem_a).wait()
        pltpu.async_copy(g_hbm.at[src_v.at[jb]], rows_b, sem_b)
        process(ja, rows_a)
        pltpu.make_async_copy(g_hbm.at[src_v.at[jb]], rows_b, sem_b).wait()

        @pl.when(jb + 1 < NCHA)
        def _():
            pltpu.async_copy(g_hbm.at[src_v.at[jb + 1]], rows_a, sem_a)

        process(jb, rows_b)
        return 0

    lax.fori_loop(0, NCHA // 2, pair, 0, unroll=False)
    pltpu.sync_copy(acc_f.at[pl.ds(0, RPT * D)], out_hbm.at[w])


def _tc1_body(deg_ref, x_ref, w1_ref, g_ref, dinv_ref):
    deg = deg_ref[...] + 1.0
    dinv = lax.rsqrt(deg)
    rows = lax.broadcasted_iota(jnp.int32, (NP, 1), 0)
    dinv = jnp.where(rows < N, dinv, 0.0)
    dinv_ref[...] = dinv
    h = jnp.dot(x_ref[...], w1_ref[...], preferred_element_type=jnp.float32)
    g_ref[...] = h * dinv


_tc1 = pl.pallas_call(
    _tc1_body,
    out_shape=(
        jax.ShapeDtypeStruct((NP, D), jnp.float32),
        jax.ShapeDtypeStruct((NP, 1), jnp.float32),
    ),
)


def _tc2_body(s_ref, g_ref, dinv_ref, b_ref, gam_ref, bet_ref, w2_ref, out_ref):
    dinv = dinv_ref[...]
    h = dinv * (s_ref[...] + g_ref[...]) + b_ref[...]
    rows = lax.broadcasted_iota(jnp.int32, (NP, 1), 0)
    m = rows < N
    hm = jnp.where(m, h, 0.0)
    mean = jnp.sum(hm, axis=0, keepdims=True) * (1.0 / N)
    cen = h - mean
    cenm = jnp.where(m, cen, 0.0)
    var = jnp.sum(cenm * cenm, axis=0, keepdims=True) * (1.0 / N)
    hbn = cen * lax.rsqrt(var + EPS) * gam_ref[...] + bet_ref[...]
    hr = jnp.maximum(hbn, 0.0)
    h2 = jnp.dot(hr, w2_ref[...], preferred_element_type=jnp.float32)
    out_ref[...] = jnp.where(m, h2 * dinv, 0.0)


_tc2 = pl.pallas_call(
    _tc2_body,
    out_shape=jax.ShapeDtypeStruct((NP, D), jnp.float32),
)


def _tc3_body(s_ref, g_ref, dinv_ref, b_ref, gam_ref, bet_ref,
              wm1_ref, bm1_ref, wm2_ref, bm2_ref, out_ref):
    dinv = dinv_ref[...]
    h = dinv * (s_ref[...] + g_ref[...]) + b_ref[...]
    rows = lax.broadcasted_iota(jnp.int32, (NP, 1), 0)
    m = rows < N
    hm = jnp.where(m, h, 0.0)
    mean = jnp.sum(hm, axis=0, keepdims=True) * (1.0 / N)
    cen = h - mean
    cenm = jnp.where(m, cen, 0.0)
    var = jnp.sum(cenm * cenm, axis=0, keepdims=True) * (1.0 / N)
    hbn = cen * lax.rsqrt(var + EPS) * gam_ref[...] + bet_ref[...]
    hr = jnp.maximum(hbn, 0.0)
    z = jnp.dot(hr, wm1_ref[...], preferred_element_type=jnp.float32)
    z = jnp.maximum(z + bm1_ref[...], 0.0)
    y = jnp.dot(z, wm2_ref[...], preferred_element_type=jnp.float32)
    out_ref[...] = y + bm2_ref[...]


_tc3 = pl.pallas_call(
    _tc3_body,
    out_shape=jax.ShapeDtypeStruct((NP, 1), jnp.float32),
)


def kernel(x, edge_index, W1, b1, gamma1, beta1, W2, b2, gamma2, beta2,
           Wm1, bm1, Wm2, bm2):
    ei = edge_index.astype(jnp.int32)
    src, dst = ei[0], ei[1]
    x_p = jnp.pad(x, ((0, NP - N), (0, 0)))
    zflat = jnp.zeros((16384,), jnp.float32)

    srcl, offl, degt = _sc_route(src, dst)
    deg = degt[:, :RPT].reshape(NP, 1)
    srcl3 = srcl.reshape(32, NCHA, CH)

    g1, dinv = _tc1(deg, x_p, W1)
    s1 = _sc_accum(srcl3, offl, g1, zflat).reshape(NP, D)
    g2 = _tc2(s1, g1, dinv, b1.reshape(1, D), gamma1.reshape(1, D),
              beta1.reshape(1, D), W2)
    s2 = _sc_accum(srcl3, offl, g2, zflat).reshape(NP, D)
    y = _tc3(s2, g2, dinv, b2.reshape(1, D), gamma2.reshape(1, D),
             beta2.reshape(1, D), Wm1, bm1.reshape(1, 64), Wm2,
             bm2.reshape(1, 1))
    return y[:N, 0]
